# baseline (speedup 1.0000x reference)
import functools

import jax
import jax.numpy as jnp
import numpy as np
from jax import lax
from jax.experimental import pallas as pl
from jax.experimental.pallas import tpu as pltpu
from jax.experimental.pallas import tpu_sc as plsc

B = 4096
C = 1000
BLK = 512
GRID = B // BLK

NC = 2
NS = 16
NW = NC * NS
RPT = B // NW

def _gen_offset():
    return jax.random.randint(jax.random.key(42), (B,), 1, C, dtype=jnp.int32)


try:
    try:
        with jax.default_device(jax.local_devices(backend="cpu")[0]):
            _OFFSET = np.asarray(_gen_offset())
    except Exception:
        _OFFSET = np.asarray(_gen_offset())
except Exception:
    _OFFSET = None


def _fixed_offset():
    return _gen_offset() if _OFFSET is None else jnp.asarray(_OFFSET)


_sc_mesh = plsc.VectorSubcoreMesh(core_axis_name="c", subcore_axis_name="s")


@functools.partial(
    pl.kernel,
    mesh=_sc_mesh,
    out_type=[
        jax.ShapeDtypeStruct((B,), jnp.float32),
        jax.ShapeDtypeStruct((B,), jnp.float32),
    ],
    scratch_types=[
        pltpu.VMEM((16, C), jnp.float32),
        pltpu.VMEM((RPT,), jnp.int32),
        pltpu.VMEM((RPT,), jnp.int32),
        pltpu.VMEM((RPT,), jnp.float32),
        pltpu.VMEM((RPT,), jnp.float32),
    ],
    compiler_params=pltpu.CompilerParams(
        use_tc_tiling_on_sc=True, needs_layout_passes=False
    ),
)
def _sc_gather(x_hbm, tgt_hbm, off_hbm, oy_hbm, ok_hbm,
               xt_v, tgt_v, off_v, oy_v, ok_v):
    wid = lax.axis_index("s") * NC + lax.axis_index("c")
    base = wid * RPT
    pltpu.sync_copy(tgt_hbm.at[pl.ds(base, RPT)], tgt_v)
    pltpu.sync_copy(off_hbm.at[pl.ds(base, RPT)], off_v)
    r = lax.iota(jnp.int32, 16)
    for j in range(RPT // 16):
        pltpu.sync_copy(x_hbm.at[pl.ds(base + j * 16, 16)], xt_v)
        t = tgt_v[pl.ds(j * 16, 16)]
        o = off_v[pl.ds(j * 16, 16)]
        n = lax.rem(t + o, C)
        oy_v[pl.ds(j * 16, 16)] = plsc.load_gather(xt_v, [r, t])
        ok_v[pl.ds(j * 16, 16)] = plsc.load_gather(xt_v, [r, n])
    pltpu.sync_copy(oy_v, oy_hbm.at[pl.ds(base, RPT)])
    pltpu.sync_copy(ok_v, ok_hbm.at[pl.ds(base, RPT)])


def _lse_body(x_ref, lz_ref):
    x = x_ref[...]
    m = jnp.max(x, axis=1, keepdims=True)
    z = jnp.sum(jnp.exp(x - m), axis=1, keepdims=True)
    lz_ref[...] = m + jnp.log(z)


def _loss_body(lz_ref, oy_ref, ok_ref, out_ref):
    lz = lz_ref[...]
    oy = oy_ref[...]
    ok = ok_ref[...]
    wy = jnp.clip(jnp.exp(oy - lz), 1e-7, 1.0)
    wk = jnp.clip(jnp.exp(ok - lz), 1e-7, 1.0)
    tt = 1.0 - (wk - wy)
    gneg = -(wk * (wy + wk)) * tt - wk * (1.0 - wk) * tt
    gpos = wk * tt + wk * wy * tt
    out_ref[...] = (-jnp.sum(gneg * ok + gpos * oy) / B).reshape(1, 1)


def kernel(output, target):
    oy, ok = _sc_gather(output, target, jnp.asarray(_fixed_offset()))
    lz = pl.pallas_call(
        _lse_body,
        grid=(GRID,),
        in_specs=[pl.BlockSpec((BLK, C), lambda i: (i, 0))],
        out_specs=pl.BlockSpec((BLK, 1), lambda i: (i, 0)),
        out_shape=jax.ShapeDtypeStruct((B, 1), jnp.float32),
    )(output)
    out = pl.pallas_call(
        _loss_body,
        in_specs=[
            pl.BlockSpec((32, 128), lambda: (0, 0)),
            pl.BlockSpec((32, 128), lambda: (0, 0)),
            pl.BlockSpec((32, 128), lambda: (0, 0)),
        ],
        out_specs=pl.BlockSpec((1, 1), lambda: (0, 0)),
        out_shape=jax.ShapeDtypeStruct((1, 1), jnp.float32),
    )(lz.reshape(32, 128), oy.reshape(32, 128), ok.reshape(32, 128))
    return out[0, 0]


# speedup vs baseline: 1.4074x; 1.4074x over previous
import jax
import jax.numpy as jnp
import numpy as np
from jax import lax
from jax.experimental import pallas as pl

B = 4096
C = 1000
BLK = 256
GRID = B // BLK

_OFFSET = np.asarray(
    jax.random.randint(jax.random.key(42), (B,), 1, C, dtype=jnp.int32)
)

_NWIN = 7


def _gather128(x, idx):
    win = jnp.minimum(idx >> 7, _NWIN)
    acc = jnp.zeros((BLK, 128), jnp.float32)
    for k in range(_NWIN + 1):
        start = 128 * k if k < _NWIN else C - 128
        acc = jnp.where(win == k, x[:, start:start + 128], acc)
    start_of = jnp.where(win == _NWIN, C - 128, win << 7)
    lane = idx - start_of
    cols = lax.broadcasted_iota(jnp.int32, (BLK, 128), 1)
    return jnp.sum(jnp.where(cols == lane, acc, 0.0), axis=1, keepdims=True)


def _body(x_ref, t_ref, o_ref, out_ref):
    i = pl.program_id(0)
    x = x_ref[...]
    t = t_ref[0, 0, :].reshape(BLK, 1)
    n = lax.rem(t + o_ref[0, 0, :].reshape(BLK, 1), C)

    m = jnp.max(x, axis=1, keepdims=True)
    z = jnp.sum(jnp.exp(x - m), axis=1, keepdims=True)
    lz = m + jnp.log(z)

    oy = _gather128(x, t)
    ok = _gather128(x, n)
    wy = jnp.clip(jnp.exp(oy - lz), 1e-7, 1.0)
    wk = jnp.clip(jnp.exp(ok - lz), 1e-7, 1.0)

    tt = 1.0 - (wk - wy)
    gneg = -(wk * (wy + wk)) * tt - wk * (1.0 - wk) * tt
    gpos = wk * tt + wk * wy * tt
    partial = jnp.sum(gneg * ok + gpos * oy).reshape(1, 1)

    prev = jnp.where(i == 0, jnp.zeros((1, 1), jnp.float32), out_ref[...])
    tot = prev + partial
    out_ref[...] = jnp.where(i == GRID - 1, -tot / B, tot)


def kernel(output, target):
    offset3 = jnp.asarray(_OFFSET).reshape(GRID, 1, BLK)
    out = pl.pallas_call(
        _body,
        grid=(GRID,),
        in_specs=[
            pl.BlockSpec((BLK, C), lambda i: (i, 0)),
            pl.BlockSpec((1, 1, BLK), lambda i: (i, 0, 0)),
            pl.BlockSpec((1, 1, BLK), lambda i: (i, 0, 0)),
        ],
        out_specs=pl.BlockSpec((1, 1), lambda i: (0, 0)),
        out_shape=jax.ShapeDtypeStruct((1, 1), jnp.float32),
    )(output, target.reshape(GRID, 1, BLK), offset3)
    return out[0, 0]


# speedup vs baseline: 1.5383x; 1.0930x over previous
import jax
import jax.numpy as jnp
import numpy as np
from jax import lax
from jax.experimental import pallas as pl

B = 4096
C = 1000
BLK = 1024
GRID = B // BLK

_OFFSET = np.asarray(
    jax.random.randint(jax.random.key(42), (B,), 1, C, dtype=jnp.int32)
)

_NWIN = 7


def _gather128(x, idx):
    win = jnp.minimum(idx >> 7, _NWIN)
    acc = jnp.zeros((BLK, 128), jnp.float32)
    for k in range(_NWIN + 1):
        start = 128 * k if k < _NWIN else C - 128
        acc = jnp.where(win == k, x[:, start:start + 128], acc)
    start_of = jnp.where(win == _NWIN, C - 128, win << 7)
    lane = idx - start_of
    cols = lax.broadcasted_iota(jnp.int32, (BLK, 128), 1)
    return jnp.sum(jnp.where(cols == lane, acc, 0.0), axis=1, keepdims=True)


def _body(x_ref, t_ref, o_ref, out_ref):
    i = pl.program_id(0)
    x = x_ref[...]
    t = t_ref[0, 0, :].reshape(BLK, 1)
    n = lax.rem(t + o_ref[0, 0, :].reshape(BLK, 1), C)

    m = jnp.max(x, axis=1, keepdims=True)
    z = jnp.sum(jnp.exp(x - m), axis=1, keepdims=True)
    lz = m + jnp.log(z)

    oy = _gather128(x, t)
    ok = _gather128(x, n)
    wy = jnp.clip(jnp.exp(oy - lz), 1e-7, 1.0)
    wk = jnp.clip(jnp.exp(ok - lz), 1e-7, 1.0)

    tt = 1.0 - (wk - wy)
    gneg = -(wk * (wy + wk)) * tt - wk * (1.0 - wk) * tt
    gpos = wk * tt + wk * wy * tt
    partial = jnp.sum(gneg * ok + gpos * oy).reshape(1, 1)

    prev = jnp.where(i == 0, jnp.zeros((1, 1), jnp.float32), out_ref[...])
    tot = prev + partial
    out_ref[...] = jnp.where(i == GRID - 1, -tot / B, tot)


def kernel(output, target):
    offset3 = jnp.asarray(_OFFSET).reshape(GRID, 1, BLK)
    out = pl.pallas_call(
        _body,
        grid=(GRID,),
        in_specs=[
            pl.BlockSpec((BLK, C), lambda i: (i, 0)),
            pl.BlockSpec((1, 1, BLK), lambda i: (i, 0, 0)),
            pl.BlockSpec((1, 1, BLK), lambda i: (i, 0, 0)),
        ],
        out_specs=pl.BlockSpec((1, 1), lambda i: (0, 0)),
        out_shape=jax.ShapeDtypeStruct((1, 1), jnp.float32),
    )(output, target.reshape(GRID, 1, BLK), offset3)
    return out[0, 0]
